# baseline (speedup 1.0000x reference)
import functools
import numpy as np
import jax, jax.numpy as jnp
from jax.experimental import pallas as pl
from jax.experimental.pallas import tpu as pltpu

D = 128
N = 200
RESO = 50000.0
CB = 256

_DIV = np.exp(np.arange(0, D, 2).astype(np.float32) * (-np.log(10000.0) / D))
_OMEGA = np.zeros((D,), np.float32)
_OMEGA[0::2] = _DIV
_OMEGA[1::2] = _DIV
_PHASE = np.where(np.arange(D) % 2 == 1, np.float32(np.pi / 2),
                  np.float32(0.0)).astype(np.float32)
_AUX = np.stack([_OMEGA, _PHASE], axis=1)


def _tc_body(locT_ref, intT_ref, aux_ref, outT_ref):
    omega = aux_ref[:, 0][:, None]
    phase = aux_ref[:, 1][:, None]

    def peak(i, acc):
        lf = jnp.ceil(locT_ref[pl.ds(i, 1), :] * RESO)
        w = jnp.where(lf == 0.0, 0.0, intT_ref[pl.ds(i, 1), :])
        ang = omega * lf
        s = jnp.sin(ang + phase)
        return acc + w * s

    acc = jax.lax.fori_loop(0, N, peak, jnp.zeros((D, CB), jnp.float32))
    outT_ref[...] = acc


@jax.jit
def tc_encode(loc, inten):
    b = loc.shape[0]
    outT = pl.pallas_call(
        _tc_body,
        out_shape=jax.ShapeDtypeStruct((D, b), jnp.float32),
        grid=(b // CB,),
        in_specs=[pl.BlockSpec((N, CB), lambda i: (0, i)),
                  pl.BlockSpec((N, CB), lambda i: (0, i)),
                  pl.BlockSpec((D, 2), lambda i: (0, 0))],
        out_specs=pl.BlockSpec((D, CB), lambda i: (0, i)),
    )(loc.T, inten.T, jnp.asarray(_AUX))
    return outT.T


def kernel(peaks_location, peaks_intensity, pe):
    return tc_encode(peaks_location, peaks_intensity)


# speedup vs baseline: 4.7166x; 4.7166x over previous
import functools

import numpy as np
import jax
import jax.numpy as jnp
from jax import lax
from jax.experimental import pallas as pl
from jax.experimental.pallas import tpu as pltpu
from jax.experimental.pallas import tpu_sc as plsc

D = 128
N = 200
B = 1024
RESO = 50000.0
NC, NS, L = 2, 16, 16
NW = NC * NS

TC_ROWS = 128
CB = 128

B_SC = B - TC_ROWS
RPW = B_SC // NW
PPW = RPW // 2
PAIR = 2 * N
GROUPS = PAIR // L
CVECS = D // L
CHUNKS = ((0, 104), (104, 96))


def _sc_body(loc_hbm, int_hbm, pe_hbm, out_hbm,
             locv, intv, idxv, outv, rows0, rows1, rows2, sem0, sem1, sem2):
    wid = lax.axis_index("s") * NC + lax.axis_index("c")
    base = wid * (PPW * PAIR)

    pltpu.sync_copy(loc_hbm.at[pl.ds(base, PPW * PAIR)], locv)
    pltpu.sync_copy(int_hbm.at[pl.ds(base, PPW * PAIR)],
                    intv.at[pl.ds(0, PPW * PAIR)])

    def idx_body(k, carry):
        t = locv[pl.ds(k * L, L)] * RESO
        f = t.astype(jnp.int32)
        idxv[pl.ds(k * L, L)] = jnp.where(f.astype(jnp.float32) < t, f + 1, f)
        return carry

    lax.fori_loop(0, PPW * GROUPS, idx_body, 0)

    def fire(r, rows, sem):
        for off, sz in CHUNKS:
            pltpu.async_copy(
                pe_hbm.at[idxv.at[pl.ds(r * N + off, sz)]],
                rows.at[pl.ds(off, sz)], sem)

    def drain(rows, sem):
        for off, sz in CHUNKS:
            pltpu.make_async_copy(
                pe_hbm.at[pl.ds(0, sz)], rows.at[pl.ds(off, sz)], sem).wait()

    splats = [jnp.full((L, 1), i, jnp.int32) for i in range(L)]
    bcast_dnums = lax.GatherDimensionNumbers(
        offset_dims=(), collapsed_slice_dims=(0,), start_index_map=(0,))

    def lane_bcast(vec, i):
        return lax.gather(vec, splats[i], bcast_dnums, slice_sizes=(1,),
                          mode=lax.GatherScatterMode.PROMISE_IN_BOUNDS)

    def compute(r, rows):
        ibase = r * N
        zeros = tuple(jnp.zeros((L,), jnp.float32) for _ in range(CVECS))

        def block(g, accs):
            res = list(accs)
            wvec = intv[pl.ds(ibase + g * L, L)]
            for i in range(L):
                wb = lane_bcast(wvec, i)
                pk = g * L + i
                for c in range(CVECS):
                    res[c] = res[c] + wb * rows[pk, pl.ds(c * L, L)]
            return tuple(res)

        acc = lax.fori_loop(0, N // L, block, zeros)
        wvec = intv[pl.ds(ibase + (N // L) * L, L)]
        acc = list(acc)
        for i in range(N - (N // L) * L):
            wb = lane_bcast(wvec, i)
            pk = (N // L) * L + i
            for c in range(CVECS):
                acc[c] = acc[c] + wb * rows[pk, pl.ds(c * L, L)]

        for c in range(CVECS):
            outv[pl.ds(r * D + c * L, L)] = acc[c]

    bufs = ((rows0, sem0), (rows1, sem1), (rows2, sem2))
    fire(0, rows0, sem0)
    fire(1, rows1, sem1)

    def step(j, carry):
        r0 = 3 * j
        for k in range(3):
            rows, sem = bufs[k]

            @pl.when(r0 + k + 2 < RPW)
            def _():
                nrows, nsem = bufs[(k + 2) % 3]
                fire(r0 + k + 2, nrows, nsem)

            drain(rows, sem)
            compute(r0 + k, rows)
        return carry

    lax.fori_loop(0, RPW // 3, step, 0)
    for r in range(3 * (RPW // 3), RPW):
        rows, sem = bufs[r % 3]
        drain(rows, sem)
        compute(r, rows)

    pltpu.sync_copy(outv, out_hbm.at[pl.ds(wid * (RPW * D), RPW * D)])


@jax.jit
def _sc_encode(loc_flat, int_flat, pe):
    mesh = plsc.VectorSubcoreMesh(
        core_axis_name="c", subcore_axis_name="s",
        num_cores=NC, num_subcores=NS)
    f = functools.partial(
        pl.kernel,
        out_type=jax.ShapeDtypeStruct((B_SC * D,), jnp.float32),
        mesh=mesh,
        scratch_types=[
            pltpu.VMEM((PPW * PAIR,), jnp.float32),
            pltpu.VMEM((PPW * PAIR + L,), jnp.float32),
            pltpu.VMEM((PPW * PAIR,), jnp.int32),
            pltpu.VMEM((RPW * D,), jnp.float32),
            pltpu.VMEM((N, D), jnp.float32),
            pltpu.VMEM((N, D), jnp.float32),
            pltpu.VMEM((N, D), jnp.float32),
            pltpu.SemaphoreType.DMA,
            pltpu.SemaphoreType.DMA,
            pltpu.SemaphoreType.DMA,
        ],
    )(_sc_body)
    return f(loc_flat, int_flat, pe)


_DIV = np.exp(np.arange(0, D, 2).astype(np.float32) * (-np.log(10000.0) / D))
_OMEGA = np.zeros((D,), np.float32)
_OMEGA[0::2] = _DIV
_OMEGA[1::2] = _DIV
_PHASE = np.where(np.arange(D) % 2 == 1, np.float32(np.pi / 2),
                  np.float32(0.0)).astype(np.float32)
_AUX = np.stack([_OMEGA, _PHASE], axis=1)


def _tc_body(locT_ref, intT_ref, aux_ref, outT_ref):
    omega = aux_ref[:, 0][:, None]
    phase = aux_ref[:, 1][:, None]

    def peak(i, acc):
        lf = jnp.ceil(locT_ref[pl.ds(i, 1), :] * RESO)
        w = jnp.where(lf == 0.0, 0.0, intT_ref[pl.ds(i, 1), :])
        ang = omega * lf
        s = jnp.sin(ang + phase)
        return acc + w * s

    acc = lax.fori_loop(0, N, peak, jnp.zeros((D, CB), jnp.float32))
    outT_ref[...] = acc


def _tc_encode(locT, intT):
    outT = pl.pallas_call(
        _tc_body,
        out_shape=jax.ShapeDtypeStruct((D, TC_ROWS), jnp.float32),
        grid=(TC_ROWS // CB,),
        in_specs=[pl.BlockSpec((N, CB), lambda i: (0, i)),
                  pl.BlockSpec((N, CB), lambda i: (0, i)),
                  pl.BlockSpec((D, 2), lambda i: (0, 0))],
        out_specs=pl.BlockSpec((D, CB), lambda i: (0, i)),
    )(locT, intT, jnp.asarray(_AUX))
    return outT.T


@jax.jit
def _encode(peaks_location, peaks_intensity, pe):
    out_sc = _sc_encode(peaks_location[TC_ROWS:].reshape(-1),
                        peaks_intensity[TC_ROWS:].reshape(-1), pe)
    out_tc = _tc_encode(peaks_location[:TC_ROWS].T,
                        peaks_intensity[:TC_ROWS].T)
    return jnp.concatenate([out_tc, out_sc.reshape(B_SC, D)], axis=0)


def kernel(peaks_location, peaks_intensity, pe):
    return _encode(peaks_location, peaks_intensity, pe)


# speedup vs baseline: 5.7294x; 1.2147x over previous
import functools

import jax
import jax.numpy as jnp
from jax import lax
from jax.experimental import pallas as pl
from jax.experimental.pallas import tpu as pltpu
from jax.experimental.pallas import tpu_sc as plsc

D = 128
N = 200
B = 1024
RESO = 50000.0
NC, NS, L = 2, 16, 16
NW = NC * NS
PAIR = 2 * N
NPAIRS = B // 2
PPW = NPAIRS // NW
RPW = B // NW
GROUPS = PAIR // L
CVECS = D // L
CHUNKS = ((0, 104), (104, 96))


def _sc_body(loc_hbm, int_hbm, pe_hbm, out_hbm,
             locv, intv, idxv, outv, rows0, rows1, rows2, sem0, sem1, sem2):
    wid = lax.axis_index("s") * NC + lax.axis_index("c")
    base = wid * (PPW * PAIR)

    pltpu.sync_copy(loc_hbm.at[pl.ds(base, PPW * PAIR)], locv)
    pltpu.sync_copy(int_hbm.at[pl.ds(base, PPW * PAIR)],
                    intv.at[pl.ds(0, PPW * PAIR)])

    def idx_body(k, carry):
        t = locv[pl.ds(k * L, L)] * RESO
        f = t.astype(jnp.int32)
        idxv[pl.ds(k * L, L)] = jnp.where(f.astype(jnp.float32) < t, f + 1, f)
        return carry

    lax.fori_loop(0, GROUPS, idx_body, 0)

    def fire(r, rows, sem):
        for off, sz in CHUNKS:
            pltpu.async_copy(
                pe_hbm.at[idxv.at[pl.ds(r * N + off, sz)]],
                rows.at[pl.ds(off, sz)], sem)

    def drain(rows, sem):
        for off, sz in CHUNKS:
            pltpu.make_async_copy(
                pe_hbm.at[pl.ds(0, sz)], rows.at[pl.ds(off, sz)], sem).wait()

    splats = [jnp.full((L, 1), i, jnp.int32) for i in range(L)]
    bcast_dnums = lax.GatherDimensionNumbers(
        offset_dims=(), collapsed_slice_dims=(0,), start_index_map=(0,))

    def lane_bcast(vec, i):
        return lax.gather(vec, splats[i], bcast_dnums, slice_sizes=(1,),
                          mode=lax.GatherScatterMode.PROMISE_IN_BOUNDS)

    def compute(r, rows):
        ibase = r * N
        zeros = tuple(jnp.zeros((L,), jnp.float32) for _ in range(CVECS))

        def block(g, accs):
            res = list(accs)
            wvec = intv[pl.ds(ibase + g * L, L)]
            for i in range(L):
                wb = lane_bcast(wvec, i)
                pk = g * L + i
                for c in range(CVECS):
                    res[c] = res[c] + wb * rows[pk, pl.ds(c * L, L)]
            return tuple(res)

        acc = lax.fori_loop(0, N // L, block, zeros)
        wvec = intv[pl.ds(ibase + (N // L) * L, L)]
        acc = list(acc)
        for i in range(N - (N // L) * L):
            wb = lane_bcast(wvec, i)
            pk = (N // L) * L + i
            for c in range(CVECS):
                acc[c] = acc[c] + wb * rows[pk, pl.ds(c * L, L)]

        for c in range(CVECS):
            outv[pl.ds(r * D + c * L, L)] = acc[c]

    bufs = ((rows0, sem0), (rows1, sem1), (rows2, sem2))
    fire(0, rows0, sem0)
    fire(1, rows1, sem1)
    lax.fori_loop(GROUPS, PPW * GROUPS, idx_body, 0)

    def step(j, carry):
        r0 = 3 * j
        for k in range(3):
            rows, sem = bufs[k]

            @pl.when(r0 + k + 2 < RPW)
            def _():
                nrows, nsem = bufs[(k + 2) % 3]
                fire(r0 + k + 2, nrows, nsem)

            drain(rows, sem)
            compute(r0 + k, rows)
        return carry

    lax.fori_loop(0, RPW // 3, step, 0)
    drain(rows0, sem0)
    compute(30, rows0)
    drain(rows1, sem1)
    compute(31, rows1)

    pltpu.sync_copy(outv, out_hbm.at[pl.ds(wid * (RPW * D), RPW * D)])


@jax.jit
def _sc_encode(loc_flat, int_flat, pe):
    mesh = plsc.VectorSubcoreMesh(
        core_axis_name="c", subcore_axis_name="s",
        num_cores=NC, num_subcores=NS)
    f = functools.partial(
        pl.kernel,
        out_type=jax.ShapeDtypeStruct((B * D,), jnp.float32),
        mesh=mesh,
        scratch_types=[
            pltpu.VMEM((PPW * PAIR,), jnp.float32),
            pltpu.VMEM((PPW * PAIR + L,), jnp.float32),
            pltpu.VMEM((PPW * PAIR,), jnp.int32),
            pltpu.VMEM((RPW * D,), jnp.float32),
            pltpu.VMEM((N, D), jnp.float32),
            pltpu.VMEM((N, D), jnp.float32),
            pltpu.VMEM((N, D), jnp.float32),
            pltpu.SemaphoreType.DMA,
            pltpu.SemaphoreType.DMA,
            pltpu.SemaphoreType.DMA,
        ],
    )(_sc_body)
    return f(loc_flat, int_flat, pe)


def kernel(peaks_location, peaks_intensity, pe):
    out = _sc_encode(peaks_location.reshape(-1), peaks_intensity.reshape(-1), pe)
    return out.reshape(B, D)
